# baseline (speedup 1.0000x reference)
import jax
import jax.numpy as jnp
from jax.experimental import pallas as pl
from jax.experimental.pallas import tpu as pltpu

_STREAMS = 4
_BLOCK = 2000


def _linear_kernel(*refs):
    xs = refs[:_STREAMS]
    w_ref = refs[_STREAMS]
    b_ref = refs[_STREAMS + 1]
    o_ref = refs[_STREAMS + 2]
    w = w_ref[...]
    bv = b_ref[...]
    for s, x_ref in enumerate(xs):
        acc = jax.lax.dot_general(
            x_ref[0], w,
            dimension_numbers=(((1,), (1,)), ((), ())),
            preferred_element_type=jnp.float32,
        )
        o_ref[s] = acc + bv


def kernel(x, W, b):
    n, k = x.shape
    c = W.shape[0]
    b2 = b.reshape(1, c)
    chunk = n // _STREAMS
    steps = chunk // _BLOCK
    x3 = x.reshape(_STREAMS, chunk, k)

    def in_spec(s):
        return pl.BlockSpec((1, _BLOCK, k), lambda i, s=s: (s, i, 0))

    out = pl.pallas_call(
        _linear_kernel,
        grid=(steps,),
        in_specs=[in_spec(s) for s in range(_STREAMS)]
        + [
            pl.BlockSpec((c, k), lambda i: (0, 0)),
            pl.BlockSpec((1, c), lambda i: (0, 0)),
        ],
        out_specs=pl.BlockSpec((_STREAMS, _BLOCK, c), lambda i: (0, i, 0)),
        out_shape=jax.ShapeDtypeStruct((_STREAMS, chunk, c), x.dtype),
        compiler_params=pltpu.CompilerParams(
            dimension_semantics=("arbitrary",),
        ),
    )(*([x3] * _STREAMS), W, b2)
    return out.reshape(n, c)
